# speedup vs baseline: 5.5204x; 5.5204x over previous
import jax
import jax.numpy as jnp
from jax.experimental import pallas as pl
from jax.experimental.pallas import tpu as pltpu

H = 4096
I = 11008
K = I // 2
IB = 256
NI = I // IB


def _up_body(x_ref, wg_ref, wu_ref, prod_ref, bits_ref):
    x = x_ref[...]
    z = jax.lax.dot_general(x, wg_ref[...], (((1,), (1,)), ((), ())),
                            preferred_element_type=jnp.float32)
    u = jax.lax.dot_general(x, wu_ref[...], (((1,), (1,)), ((), ())),
                            preferred_element_type=jnp.float32)
    gate = z * (1.0 / (1.0 + jnp.exp(-z)))
    prod_ref[...] = gate * u
    bits_ref[...] = jax.lax.bitcast_convert_type(jnp.abs(gate), jnp.int32)


def _down_body(bits_ref, prod_ref, wd_ref, out_ref, thr_ref):
    i = pl.program_id(0)

    @pl.when(i == 0)
    def _():
        bits = bits_ref[...]

        def body(b, v):
            q = v + (1 << (30 - b))
            cnt = jnp.sum((bits < q).astype(jnp.int32), axis=1, keepdims=True)
            return jnp.where(cnt >= K, v, q)

        thr_ref[...] = jax.lax.fori_loop(0, 31, body,
                                         jnp.zeros((32, 1), jnp.int32))
        out_ref[...] = jnp.zeros_like(out_ref)

    v = thr_ref[...]
    blk = bits_ref[:, pl.ds(i * IB, IB)]
    masked = jnp.where(blk > v, prod_ref[:, pl.ds(i * IB, IB)], 0.0)
    out_ref[...] += jax.lax.dot_general(masked, wd_ref[...],
                                        (((1,), (1,)), ((), ())),
                                        preferred_element_type=jnp.float32)


def kernel(x, Wg, Wu, Wd):
    B = x.shape[0]
    x2 = x.reshape(B, H)

    prod, bits = pl.pallas_call(
        _up_body,
        grid=(NI,),
        in_specs=[
            pl.BlockSpec((B, H), lambda i: (0, 0)),
            pl.BlockSpec((IB, H), lambda i: (i, 0)),
            pl.BlockSpec((IB, H), lambda i: (i, 0)),
        ],
        out_specs=[
            pl.BlockSpec((B, IB), lambda i: (0, i)),
            pl.BlockSpec((B, IB), lambda i: (0, i)),
        ],
        out_shape=[
            jax.ShapeDtypeStruct((B, I), jnp.float32),
            jax.ShapeDtypeStruct((B, I), jnp.int32),
        ],
    )(x2, Wg, Wu)

    out = pl.pallas_call(
        _down_body,
        grid=(NI,),
        in_specs=[
            pl.BlockSpec((B, I), lambda i: (0, 0)),
            pl.BlockSpec((B, I), lambda i: (0, 0)),
            pl.BlockSpec((H, IB), lambda i: (0, i)),
        ],
        out_specs=pl.BlockSpec((B, H), lambda i: (0, 0)),
        out_shape=jax.ShapeDtypeStruct((B, H), jnp.float32),
        scratch_shapes=[pltpu.VMEM((32, 1), jnp.int32)],
    )(bits, prod, Wd)

    return out.reshape(B, 1, H)


# speedup vs baseline: 5.8761x; 1.0644x over previous
import jax
import jax.numpy as jnp
from jax.experimental import pallas as pl
from jax.experimental.pallas import tpu as pltpu

H = 4096
I = 11008
K = I // 2
IB = 256
NI = I // IB
HB = 512
NH = H // HB


def _up_body(x_ref, wg_ref, wu_ref, prod_ref, bits_ref):
    x = x_ref[...]
    z = jax.lax.dot_general(x, wg_ref[...], (((1,), (1,)), ((), ())),
                            preferred_element_type=jnp.float32)
    u = jax.lax.dot_general(x, wu_ref[...], (((1,), (1,)), ((), ())),
                            preferred_element_type=jnp.float32)
    gate = z * (1.0 / (1.0 + jnp.exp(-z)))
    prod_ref[...] = gate * u
    bits_ref[...] = jax.lax.bitcast_convert_type(jnp.abs(gate), jnp.int32)


def _down_body(bits_ref, prod_ref, wd_ref, out_ref, masked_ref):
    i = pl.program_id(0)

    @pl.when(i == 0)
    def _():
        bits = bits_ref[...]

        def body(b, v):
            q = v + (1 << (30 - b))
            cnt = jnp.sum((bits < q).astype(jnp.int32), axis=1, keepdims=True)
            return jnp.where(cnt >= K, v, q)

        v = jax.lax.fori_loop(0, 31, body, jnp.zeros((32, 1), jnp.int32))
        masked_ref[...] = jnp.where(bits > v, prod_ref[...], 0.0)

    out_ref[...] = jax.lax.dot_general(masked_ref[...], wd_ref[...],
                                       (((1,), (1,)), ((), ())),
                                       preferred_element_type=jnp.float32)


def kernel(x, Wg, Wu, Wd):
    B = x.shape[0]
    x2 = x.reshape(B, H)

    prod, bits = pl.pallas_call(
        _up_body,
        grid=(NI,),
        in_specs=[
            pl.BlockSpec((B, H), lambda i: (0, 0)),
            pl.BlockSpec((IB, H), lambda i: (i, 0)),
            pl.BlockSpec((IB, H), lambda i: (i, 0)),
        ],
        out_specs=[
            pl.BlockSpec((B, IB), lambda i: (0, i)),
            pl.BlockSpec((B, IB), lambda i: (0, i)),
        ],
        out_shape=[
            jax.ShapeDtypeStruct((B, I), jnp.float32),
            jax.ShapeDtypeStruct((B, I), jnp.int32),
        ],
    )(x2, Wg, Wu)

    out = pl.pallas_call(
        _down_body,
        grid=(NH,),
        in_specs=[
            pl.BlockSpec((B, I), lambda i: (0, 0)),
            pl.BlockSpec((B, I), lambda i: (0, 0)),
            pl.BlockSpec((HB, I), lambda i: (i, 0)),
        ],
        out_specs=pl.BlockSpec((B, HB), lambda i: (0, i)),
        out_shape=jax.ShapeDtypeStruct((B, H), jnp.float32),
        scratch_shapes=[pltpu.VMEM((B, I), jnp.float32)],
    )(bits, prod, Wd)

    return out.reshape(B, 1, H)


# speedup vs baseline: 6.1085x; 1.0395x over previous
import jax
import jax.numpy as jnp
from jax.experimental import pallas as pl
from jax.experimental.pallas import tpu as pltpu

H = 4096
I = 11008
K = I // 2
IB = 512
NI = (I + IB - 1) // IB
HB = 512
NH = H // HB


def _up_body(x_ref, wg_ref, wu_ref, prod_ref, bits_ref):
    x = x_ref[...]
    z = jax.lax.dot_general(x, wg_ref[...], (((1,), (1,)), ((), ())),
                            preferred_element_type=jnp.float32)
    u = jax.lax.dot_general(x, wu_ref[...], (((1,), (1,)), ((), ())),
                            preferred_element_type=jnp.float32)
    gate = z * (1.0 / (1.0 + jnp.exp(-z)))
    prod_ref[...] = gate * u
    bits_ref[...] = jax.lax.bitcast_convert_type(jnp.abs(gate), jnp.int32)


def _down_body(bits_ref, prod_ref, wd_ref, out_ref, masked_ref):
    i = pl.program_id(0)

    @pl.when(i == 0)
    def _():
        bits = bits_ref[...]

        def body(b, v):
            q = v + (1 << (30 - b))
            cnt = jnp.sum((bits < q).astype(jnp.int32), axis=1, keepdims=True)
            return jnp.where(cnt >= K, v, q)

        v = jax.lax.fori_loop(0, 31, body, jnp.zeros((32, 1), jnp.int32))
        masked_ref[...] = jnp.where(bits > v, prod_ref[...], 0.0)

    out_ref[...] = jax.lax.dot_general(masked_ref[...], wd_ref[...],
                                       (((1,), (1,)), ((), ())),
                                       preferred_element_type=jnp.float32)


def kernel(x, Wg, Wu, Wd):
    B = x.shape[0]
    x2 = x.reshape(B, H)

    prod, bits = pl.pallas_call(
        _up_body,
        grid=(NI,),
        in_specs=[
            pl.BlockSpec((B, H), lambda i: (0, 0)),
            pl.BlockSpec((IB, H), lambda i: (i, 0)),
            pl.BlockSpec((IB, H), lambda i: (i, 0)),
        ],
        out_specs=[
            pl.BlockSpec((B, IB), lambda i: (0, i)),
            pl.BlockSpec((B, IB), lambda i: (0, i)),
        ],
        out_shape=[
            jax.ShapeDtypeStruct((B, I), jnp.float32),
            jax.ShapeDtypeStruct((B, I), jnp.int32),
        ],
    )(x2, Wg, Wu)

    out = pl.pallas_call(
        _down_body,
        grid=(NH,),
        in_specs=[
            pl.BlockSpec((B, I), lambda i: (0, 0)),
            pl.BlockSpec((B, I), lambda i: (0, 0)),
            pl.BlockSpec((HB, I), lambda i: (i, 0)),
        ],
        out_specs=pl.BlockSpec((B, HB), lambda i: (0, i)),
        out_shape=jax.ShapeDtypeStruct((B, H), jnp.float32),
        scratch_shapes=[pltpu.VMEM((B, I), jnp.float32)],
    )(bits, prod, Wd)

    return out.reshape(B, 1, H)
